# baseline (speedup 1.0000x reference)
import functools

import jax
import jax.numpy as jnp
from jax.experimental import pallas as pl
from jax.experimental.pallas import tpu as pltpu


def _body(x_ref, wp_ref, wct_ref, b_ref, out_ref, acc_ref, w_ref,
          *, inv_s, nsteps):
    s = pl.program_id(0)
    part = jnp.sum(x_ref[...], axis=0)

    @pl.when(s == 0)
    def _init():
        acc_ref[...] = part
        w_ref[...] = jax.lax.dot_general(
            wp_ref[...], wct_ref[...],
            (((1,), (1,)), ((), ())),
            preferred_element_type=jnp.float32,
        ) * inv_s

    @pl.when(s != 0)
    def _accum():
        acc_ref[...] += part

    @pl.when(s == nsteps - 1)
    def _finish():
        logits = jnp.dot(
            acc_ref[...], w_ref[...], preferred_element_type=jnp.float32
        )
        out_ref[...] = jnp.transpose(logits + b_ref[...])


def kernel(x_nchw, w_proj, w_cls, b_cls):
    B, C, H, W = x_nchw.shape
    S = H * W
    NCLS = w_cls.shape[1]
    FD = w_proj.shape[1]

    xs = jnp.transpose(x_nchw, (2, 3, 0, 1)).reshape(S, B, C)
    wct = jnp.transpose(w_cls)

    SB = 49
    while S % SB:
        SB = S if SB > S else SB + 1
    nsteps = S // SB

    bias = b_cls.astype(jnp.float32).reshape(1, NCLS)

    out = pl.pallas_call(
        functools.partial(_body, inv_s=1.0 / float(S), nsteps=nsteps),
        out_shape=jax.ShapeDtypeStruct((NCLS, B), jnp.float32),
        grid=(nsteps,),
        in_specs=[
            pl.BlockSpec((SB, B, C), lambda s: (s, 0, 0)),
            pl.BlockSpec((C, FD), lambda s: (0, 0)),
            pl.BlockSpec((NCLS, FD), lambda s: (0, 0)),
            pl.BlockSpec((1, NCLS), lambda s: (0, 0)),
        ],
        out_specs=pl.BlockSpec((NCLS, B), lambda s: (0, 0)),
        scratch_shapes=[
            pltpu.VMEM((B, C), jnp.float32),
            pltpu.VMEM((C, NCLS), jnp.float32),
        ],
        compiler_params=pltpu.CompilerParams(
            dimension_semantics=("arbitrary",),
            vmem_limit_bytes=60 << 20,
        ),
    )(xs.astype(jnp.float32), w_proj.astype(jnp.float32),
      wct.astype(jnp.float32), bias)
    return jnp.transpose(out)


# speedup vs baseline: 1.0523x; 1.0523x over previous
import functools

import jax
import jax.numpy as jnp
from jax.experimental import pallas as pl
from jax.experimental.pallas import tpu as pltpu


def _body(x_ref, wp_ref, wct_ref, b_ref, out_ref, acc_ref, w_ref,
          *, inv_s, nsteps, hb):
    s = pl.program_id(0)
    j = pl.program_id(1)
    part = jnp.sum(x_ref[...], axis=0)

    @pl.when(jnp.logical_and(s == 0, j == 0))
    def _head():
        w_ref[...] = jax.lax.dot_general(
            wp_ref[...], wct_ref[...],
            (((1,), (1,)), ((), ())),
            preferred_element_type=jnp.float32,
        ) * inv_s

    @pl.when(s == 0)
    def _init():
        acc_ref[pl.ds(j * hb, hb), :] = part

    @pl.when(s != 0)
    def _accum():
        acc_ref[pl.ds(j * hb, hb), :] += part

    @pl.when(jnp.logical_and(s == nsteps - 1, j == 1))
    def _finish():
        logits = jnp.dot(
            acc_ref[...], w_ref[...], preferred_element_type=jnp.float32
        )
        out_ref[...] = jnp.transpose(logits + b_ref[...])


def kernel(x_nchw, w_proj, w_cls, b_cls):
    B, C, H, W = x_nchw.shape
    S = H * W
    NCLS = w_cls.shape[1]
    FD = w_proj.shape[1]

    xs = jnp.transpose(x_nchw, (2, 3, 0, 1)).reshape(S, B, C)
    wct = jnp.transpose(w_cls)

    SB = 49
    while S % SB:
        SB = S if SB > S else SB + 1
    nsteps = S // SB
    hb = B // 2

    bias = b_cls.astype(jnp.float32).reshape(1, NCLS)

    out = pl.pallas_call(
        functools.partial(_body, inv_s=1.0 / float(S), nsteps=nsteps, hb=hb),
        out_shape=jax.ShapeDtypeStruct((NCLS, B), jnp.float32),
        grid=(nsteps, 2),
        in_specs=[
            pl.BlockSpec((SB, hb, C), lambda s, j: (s, j, 0)),
            pl.BlockSpec((C, FD), lambda s, j: (0, 0)),
            pl.BlockSpec((NCLS, FD), lambda s, j: (0, 0)),
            pl.BlockSpec((1, NCLS), lambda s, j: (0, 0)),
        ],
        out_specs=pl.BlockSpec((NCLS, B), lambda s, j: (0, 0)),
        scratch_shapes=[
            pltpu.VMEM((B, C), jnp.float32),
            pltpu.VMEM((C, NCLS), jnp.float32),
        ],
        compiler_params=pltpu.CompilerParams(
            dimension_semantics=("arbitrary", "arbitrary"),
            vmem_limit_bytes=60 << 20,
        ),
    )(xs.astype(jnp.float32), w_proj.astype(jnp.float32),
      wct.astype(jnp.float32), bias)
    return jnp.transpose(out)
